# baseline (speedup 1.0000x reference)
import jax
import jax.numpy as jnp
from jax.experimental import pallas as pl
from jax.experimental.pallas import tpu as pltpu

_N = 4096
_NF_IN = 16
_NF = 32
_NQE = _NF + 8
_NFE = _NF + 1
_B = 4
_BR = 1024
_BC = 1024
_NR = _N // _BR
_NC = _N // _BC
_LOG2N = 12
_SQRT_M = 5.477225575051661


def _fused(segs_ref, x_ref, segr_ref, wp1_ref, wq_ref, wk_ref,
           wv_ref, wt_ref, sc_ref, bi_ref, out_ref, h_ref, q_ref, k_ref,
           v_ref, t_ref, starts_ref):
    starts_ref[0, 0] = 0
    starts_ref[0, _B] = _N
    for b in range(1, _B):
        def _bs(i, lohi, b=b):
            lo, hi = lohi
            mid = (lo + hi) // 2
            pred = segs_ref[0, mid] < b
            return (jnp.where(pred, mid + 1, lo), jnp.where(pred, hi, mid))
        lo, _ = jax.lax.fori_loop(0, _LOG2N, _bs, (0, _N))
        starts_ref[0, b] = lo

    h = jnp.dot(x_ref[...], wp1_ref[...], preferred_element_type=jnp.float32)
    h_ref[...] = h
    iota8 = jax.lax.broadcasted_iota(jnp.int32, (_N, _NQE - _NF), 1)
    oh = jnp.where(segr_ref[...] == iota8, jnp.float32(_SQRT_M),
                   jnp.float32(0.0)).astype(jnp.bfloat16)
    q_ref[:, 0:_NF] = jnp.dot(
        h, wq_ref[...], preferred_element_type=jnp.float32).astype(jnp.bfloat16)
    q_ref[:, _NF:_NQE] = oh
    k_ref[:, 0:_NF] = jnp.dot(
        h, wk_ref[...], preferred_element_type=jnp.float32).astype(jnp.bfloat16)
    k_ref[:, _NF:_NQE] = oh
    v = jnp.dot(h, wv_ref[...], preferred_element_type=jnp.float32)
    v_ref[:, 0:_NF] = v.astype(jnp.bfloat16)
    v_ref[:, _NF:_NFE] = jnp.ones((_N, 1), jnp.bfloat16)
    wt = wt_ref[...]

    s1 = jnp.zeros((1, _NF), dtype=jnp.float32)
    s2 = jnp.zeros((1, _NF), dtype=jnp.float32)
    for r in range(_NR):
        row0 = r * _BR
        qe = q_ref[row0:row0 + _BR, :]

        def col_step(c, acc, qe=qe):
            col0 = c * _BC
            kb = k_ref[pl.ds(col0, _BC), :]
            vb = v_ref[pl.ds(col0, _BC), :]
            s = jax.lax.dot_general(qe, kb, (((1,), (1,)), ((), ())),
                                    preferred_element_type=jnp.float32)
            e = jnp.exp(s).astype(jnp.bfloat16)
            return acc + jnp.dot(e, vb, preferred_element_type=jnp.float32)

        a0 = jnp.zeros((_BR, _NFE), dtype=jnp.float32)
        sl = segs_ref[0, row0]
        sh = segs_ref[0, row0 + _BR - 1]
        c_lo = starts_ref[0, sl] // _BC
        c_hi = (starts_ref[0, sh + 1] + _BC - 1) // _BC
        acc = jax.lax.fori_loop(c_lo, c_hi, col_step, a0)
        rb = acc[:, 0:_NF] / acc[:, _NF:_NFE]
        tb = jnp.dot(rb, wt, preferred_element_type=jnp.float32)
        t_ref[pl.ds(row0, _BR), :] = tb
        s1 = s1 + jnp.sum(tb, axis=0, keepdims=True)
        s2 = s2 + jnp.sum(tb * tb, axis=0, keepdims=True)

    mean = s1 / jnp.float32(_N)
    var = s2 / jnp.float32(_N) - mean * mean
    inv = jax.lax.rsqrt(var + 1e-5) * sc_ref[...]
    bias = bi_ref[...]
    for r in range(_NR):
        row0 = r * _BR
        tb = t_ref[row0:row0 + _BR, :]
        out_ref[row0:row0 + _BR, :] = (
            h_ref[row0:row0 + _BR, :] + (tb - mean) * inv + bias)


def kernel(x, segment_ids, W_p1, W_q, W_k, W_v, W_trans, bn_scale, bn_bias):
    seg = segment_ids.astype(jnp.int32)
    segs = seg.reshape(1, _N)
    segr = seg.reshape(_N, 1)
    specs = [pl.BlockSpec(memory_space=pltpu.SMEM)] + [
        pl.BlockSpec(memory_space=pltpu.VMEM)] * 9
    return pl.pallas_call(
        _fused,
        out_shape=jax.ShapeDtypeStruct((_N, _NF), jnp.float32),
        in_specs=specs,
        scratch_shapes=[
            pltpu.VMEM((_N, _NF), jnp.float32),
            pltpu.VMEM((_N, _NQE), jnp.bfloat16),
            pltpu.VMEM((_N, _NQE), jnp.bfloat16),
            pltpu.VMEM((_N, _NFE), jnp.bfloat16),
            pltpu.VMEM((_N, _NF), jnp.float32),
            pltpu.SMEM((1, _B + 1), jnp.int32),
        ],
    )(segs, x, segr, W_p1, W_q, W_k, W_v, W_trans,
      bn_scale.reshape(1, _NF), bn_bias.reshape(1, _NF))


# speedup vs baseline: 1.0056x; 1.0056x over previous
import jax
import jax.numpy as jnp
from jax.experimental import pallas as pl
from jax.experimental.pallas import tpu as pltpu

_N = 4096
_NF_IN = 16
_NF = 32
_NQE = _NF + 8
_NFE = _NF + 1
_B = 4
_BR = 512
_BC = 2048
_NR = _N // _BR
_NC = _N // _BC
_LOG2N = 12
_SQRT_M = 5.477225575051661


def _fused(segs_ref, x_ref, segr_ref, wp1_ref, wq_ref, wk_ref,
           wv_ref, wt_ref, sc_ref, bi_ref, out_ref, h_ref, q_ref, k_ref,
           v_ref, t_ref, starts_ref):
    starts_ref[0, 0] = 0
    starts_ref[0, _B] = _N
    for b in range(1, _B):
        def _bs(i, lohi, b=b):
            lo, hi = lohi
            mid = (lo + hi) // 2
            pred = segs_ref[0, mid] < b
            return (jnp.where(pred, mid + 1, lo), jnp.where(pred, hi, mid))
        lo, _ = jax.lax.fori_loop(0, _LOG2N, _bs, (0, _N))
        starts_ref[0, b] = lo

    h = jnp.dot(x_ref[...], wp1_ref[...], preferred_element_type=jnp.float32)
    h_ref[...] = h
    iota8 = jax.lax.broadcasted_iota(jnp.int32, (_N, _NQE - _NF), 1)
    oh = jnp.where(segr_ref[...] == iota8, jnp.float32(_SQRT_M),
                   jnp.float32(0.0)).astype(jnp.bfloat16)
    q_ref[:, 0:_NF] = jnp.dot(
        h, wq_ref[...], preferred_element_type=jnp.float32).astype(jnp.bfloat16)
    q_ref[:, _NF:_NQE] = oh
    k_ref[:, 0:_NF] = jnp.dot(
        h, wk_ref[...], preferred_element_type=jnp.float32).astype(jnp.bfloat16)
    k_ref[:, _NF:_NQE] = oh
    v = jnp.dot(h, wv_ref[...], preferred_element_type=jnp.float32)
    v_ref[:, 0:_NF] = v.astype(jnp.bfloat16)
    v_ref[:, _NF:_NFE] = jnp.ones((_N, 1), jnp.bfloat16)
    wt = wt_ref[...]

    s1 = jnp.zeros((1, _NF), dtype=jnp.float32)
    s2 = jnp.zeros((1, _NF), dtype=jnp.float32)
    for r in range(_NR):
        row0 = r * _BR
        qe = q_ref[row0:row0 + _BR, :]

        def col_step(c, acc, qe=qe):
            col0 = c * _BC
            kb = k_ref[pl.ds(col0, _BC), :]
            vb = v_ref[pl.ds(col0, _BC), :]
            s = jax.lax.dot_general(qe, kb, (((1,), (1,)), ((), ())),
                                    preferred_element_type=jnp.float32)
            e = jnp.exp(s).astype(jnp.bfloat16)
            return acc + jnp.dot(e, vb, preferred_element_type=jnp.float32)

        a0 = jnp.zeros((_BR, _NFE), dtype=jnp.float32)
        sl = segs_ref[0, row0]
        sh = segs_ref[0, row0 + _BR - 1]
        c_lo = starts_ref[0, sl] // _BC
        c_hi = (starts_ref[0, sh + 1] + _BC - 1) // _BC
        acc = jax.lax.fori_loop(c_lo, c_hi, col_step, a0)
        rb = acc[:, 0:_NF] / acc[:, _NF:_NFE]
        tb = jnp.dot(rb, wt, preferred_element_type=jnp.float32)
        t_ref[pl.ds(row0, _BR), :] = tb
        s1 = s1 + jnp.sum(tb, axis=0, keepdims=True)
        s2 = s2 + jnp.sum(tb * tb, axis=0, keepdims=True)

    mean = s1 / jnp.float32(_N)
    var = s2 / jnp.float32(_N) - mean * mean
    inv = jax.lax.rsqrt(var + 1e-5) * sc_ref[...]
    bias = bi_ref[...]
    for r in range(_NR):
        row0 = r * _BR
        tb = t_ref[row0:row0 + _BR, :]
        out_ref[row0:row0 + _BR, :] = (
            h_ref[row0:row0 + _BR, :] + (tb - mean) * inv + bias)


def kernel(x, segment_ids, W_p1, W_q, W_k, W_v, W_trans, bn_scale, bn_bias):
    seg = segment_ids.astype(jnp.int32)
    segs = seg.reshape(1, _N)
    segr = seg.reshape(_N, 1)
    specs = [pl.BlockSpec(memory_space=pltpu.SMEM)] + [
        pl.BlockSpec(memory_space=pltpu.VMEM)] * 9
    return pl.pallas_call(
        _fused,
        out_shape=jax.ShapeDtypeStruct((_N, _NF), jnp.float32),
        in_specs=specs,
        scratch_shapes=[
            pltpu.VMEM((_N, _NF), jnp.float32),
            pltpu.VMEM((_N, _NQE), jnp.bfloat16),
            pltpu.VMEM((_N, _NQE), jnp.bfloat16),
            pltpu.VMEM((_N, _NFE), jnp.bfloat16),
            pltpu.VMEM((_N, _NF), jnp.float32),
            pltpu.SMEM((1, _B + 1), jnp.int32),
        ],
    )(segs, x, segr, W_p1, W_q, W_k, W_v, W_trans,
      bn_scale.reshape(1, _NF), bn_bias.reshape(1, _NF))


# speedup vs baseline: 1.0505x; 1.0447x over previous
import jax
import jax.numpy as jnp
from jax.experimental import pallas as pl
from jax.experimental.pallas import tpu as pltpu

_N = 4096
_NF_IN = 16
_NF = 32
_NQE = _NF + 8
_NFE = _NF + 1
_B = 4
_BR = 512
_BC = 1024
_NR = _N // _BR
_NC = _N // _BC
_LOG2N = 12
_SQRT_M = 5.477225575051661


def _fused(segs_ref, x_ref, segr_ref, wp1_ref, wq_ref, wk_ref,
           wv_ref, wt_ref, sc_ref, bi_ref, out_ref, h_ref, q_ref, k_ref,
           v_ref, t_ref, starts_ref):
    starts_ref[0, 0] = 0
    starts_ref[0, _B] = _N
    for b in range(1, _B):
        def _bs(i, lohi, b=b):
            lo, hi = lohi
            mid = (lo + hi) // 2
            pred = segs_ref[0, mid] < b
            return (jnp.where(pred, mid + 1, lo), jnp.where(pred, hi, mid))
        lo, _ = jax.lax.fori_loop(0, _LOG2N, _bs, (0, _N))
        starts_ref[0, b] = lo

    h = jnp.dot(x_ref[...], wp1_ref[...], preferred_element_type=jnp.float32)
    h_ref[...] = h
    iota8 = jax.lax.broadcasted_iota(jnp.int32, (_N, _NQE - _NF), 1)
    oh = jnp.where(segr_ref[...] == iota8, jnp.float32(_SQRT_M),
                   jnp.float32(0.0))
    c4 = jnp.where(iota8 == 4, jnp.float32(_SQRT_M), jnp.float32(0.0))
    q_ref[:, 0:_NF] = jnp.dot(
        h, wq_ref[...], preferred_element_type=jnp.float32).astype(jnp.bfloat16)
    q_ref[:, _NF:_NQE] = (oh + c4).astype(jnp.bfloat16)
    k_ref[:, 0:_NF] = jnp.dot(
        h, wk_ref[...], preferred_element_type=jnp.float32).astype(jnp.bfloat16)
    k_ref[:, _NF:_NQE] = (oh - c4).astype(jnp.bfloat16)
    v = jnp.dot(h, wv_ref[...], preferred_element_type=jnp.float32)
    v_ref[:, 0:_NF] = v.astype(jnp.bfloat16)
    v_ref[:, _NF:_NFE] = jnp.ones((_N, 1), jnp.bfloat16)
    wt = wt_ref[...]

    s1 = jnp.zeros((1, _NF), dtype=jnp.float32)
    s2 = jnp.zeros((1, _NF), dtype=jnp.float32)
    for r in range(_NR):
        row0 = r * _BR
        qe = q_ref[row0:row0 + _BR, :]

        def col_step(c, acc, qe=qe):
            col0 = c * _BC
            kb = k_ref[pl.ds(col0, _BC), :]
            vb = v_ref[pl.ds(col0, _BC), :]
            s = jax.lax.dot_general(qe, kb, (((1,), (1,)), ((), ())),
                                    preferred_element_type=jnp.float32)
            e = jnp.exp(s.astype(jnp.bfloat16))
            return acc + jnp.dot(e, vb, preferred_element_type=jnp.float32)

        a0 = jnp.zeros((_BR, _NFE), dtype=jnp.float32)
        sl = segs_ref[0, row0]
        sh = segs_ref[0, row0 + _BR - 1]
        c_lo = starts_ref[0, sl] // _BC
        c_hi = (starts_ref[0, sh + 1] + _BC - 1) // _BC
        acc = jax.lax.fori_loop(c_lo, c_hi, col_step, a0)
        rb = acc[:, 0:_NF] / acc[:, _NF:_NFE]
        tb = jnp.dot(rb, wt, preferred_element_type=jnp.float32)
        t_ref[pl.ds(row0, _BR), :] = tb
        s1 = s1 + jnp.sum(tb, axis=0, keepdims=True)
        s2 = s2 + jnp.sum(tb * tb, axis=0, keepdims=True)

    mean = s1 / jnp.float32(_N)
    var = s2 / jnp.float32(_N) - mean * mean
    inv = jax.lax.rsqrt(var + 1e-5) * sc_ref[...]
    bias = bi_ref[...]
    for r in range(_NR):
        row0 = r * _BR
        tb = t_ref[row0:row0 + _BR, :]
        out_ref[row0:row0 + _BR, :] = (
            h_ref[row0:row0 + _BR, :] + (tb - mean) * inv + bias)


def kernel(x, segment_ids, W_p1, W_q, W_k, W_v, W_trans, bn_scale, bn_bias):
    seg = segment_ids.astype(jnp.int32)
    segs = seg.reshape(1, _N)
    segr = seg.reshape(_N, 1)
    specs = [pl.BlockSpec(memory_space=pltpu.SMEM)] + [
        pl.BlockSpec(memory_space=pltpu.VMEM)] * 9
    return pl.pallas_call(
        _fused,
        out_shape=jax.ShapeDtypeStruct((_N, _NF), jnp.float32),
        in_specs=specs,
        scratch_shapes=[
            pltpu.VMEM((_N, _NF), jnp.float32),
            pltpu.VMEM((_N, _NQE), jnp.bfloat16),
            pltpu.VMEM((_N, _NQE), jnp.bfloat16),
            pltpu.VMEM((_N, _NFE), jnp.bfloat16),
            pltpu.VMEM((_N, _NF), jnp.float32),
            pltpu.SMEM((1, _B + 1), jnp.int32),
        ],
    )(segs, x, segr, W_p1, W_q, W_k, W_v, W_trans,
      bn_scale.reshape(1, _NF), bn_bias.reshape(1, _NF))
